# Initial kernel scaffold; baseline (speedup 1.0000x reference)
import jax
import jax.numpy as jnp
from jax.experimental import pallas as pl


def kernel(x, W, gamma, beta):
    raise NotImplementedError("write your pallas kernel here")


# speedup vs baseline: 7.1248x; 7.1248x over previous
import functools

import jax
import jax.numpy as jnp
from jax import lax
from jax.experimental import pallas as pl
from jax.experimental.pallas import tpu as pltpu
from jax.experimental.pallas import tpu_sc as plsc

K = 20
B, C, N = 16, 64, 2048
OUT = 64
ROWS = 256
NEG = -3.0e38


def _stage_a_body(x_ref, w1_ref, wv_ref, u_ref, v_ref, idx_ref):
    b = pl.program_id(0)
    r = pl.program_id(1)
    xb = x_ref[0]
    x_rows = x_ref[0, :, pl.ds(r * ROWS, ROWS)]
    dimnums = (((0,), (0,)), ((), ()))
    u_ref[0] = lax.dot_general(x_rows, w1_ref[...], dimnums,
                               preferred_element_type=jnp.float32)
    v_ref[0] = lax.dot_general(x_rows, wv_ref[...], dimnums,
                               preferred_element_type=jnp.float32)
    inner = lax.dot_general(x_rows, xb, dimnums,
                            preferred_element_type=jnp.float32)
    x2 = jnp.sum(xb * xb, axis=0, keepdims=True)
    x2_rows = jnp.sum(x_rows * x_rows, axis=0, keepdims=True)
    d = 2.0 * inner - x2 - jnp.transpose(x2_rows)
    iota = lax.broadcasted_iota(jnp.int32, (1, N), 1)
    cols = []
    for _ in range(K):
        m = jnp.max(d, axis=1, keepdims=True)
        cand = jnp.where(d >= m, iota, N)
        am = jnp.min(cand, axis=1, keepdims=True)
        cols.append(am)
        d = jnp.where(iota == am, NEG, d)
    idx_ref[0] = jnp.concatenate(cols, axis=1) + b * N


def _stage_a(x, w1t, wvt):
    grid = (B, N // ROWS)
    return pl.pallas_call(
        _stage_a_body,
        grid=grid,
        in_specs=[
            pl.BlockSpec((1, C, N), lambda b, r: (b, 0, 0)),
            pl.BlockSpec((C, OUT), lambda b, r: (0, 0)),
            pl.BlockSpec((C, OUT), lambda b, r: (0, 0)),
        ],
        out_specs=[
            pl.BlockSpec((1, ROWS, OUT), lambda b, r: (b, r, 0)),
            pl.BlockSpec((1, ROWS, OUT), lambda b, r: (b, r, 0)),
            pl.BlockSpec((1, ROWS, K), lambda b, r: (b, r, 0)),
        ],
        out_shape=[
            jax.ShapeDtypeStruct((B, N, OUT), jnp.float32),
            jax.ShapeDtypeStruct((B, N, OUT), jnp.float32),
            jax.ShapeDtypeStruct((B, N, K), jnp.int32),
        ],
    )(x, w1t, wvt)


_SC_G = 4
_SC_LANES = 16


def _stage_b_body(u_hbm, idx_hbm, m_hbm, s_hbm, q_hbm,
                  idx_v, rows_v, m_v, s_v, q_v, sem):
    info = plsc.get_sparse_core_info()
    nc = info.num_cores
    wid = lax.axis_index("s") * nc + lax.axis_index("c")
    pts_per_w = (B * N) // (nc * info.num_subcores)
    n_chunks = pts_per_w // _SC_G
    base_pt = wid * pts_per_w

    def chunk(g):
        row0 = base_pt + g * _SC_G
        pltpu.sync_copy(idx_hbm.at[pl.ds(row0 * K, _SC_G * K)], idx_v)
        pltpu.async_copy(u_hbm.at[idx_v], rows_v, sem).wait()
        for p in range(_SC_G):
            for c in range(OUT // _SC_LANES):
                sl = pl.ds(c * _SC_LANES, _SC_LANES)
                val = rows_v[p * K, sl]
                acc_m = val
                acc_s = val
                acc_q = val * val
                for j in range(1, K):
                    val = rows_v[p * K + j, sl]
                    acc_m = jnp.maximum(acc_m, val)
                    acc_s = acc_s + val
                    acc_q = acc_q + val * val
                m_v[p, sl] = acc_m
                s_v[p, sl] = acc_s
                q_v[p, sl] = acc_q
        pltpu.sync_copy(m_v, m_hbm.at[pl.ds(row0, _SC_G)])
        pltpu.sync_copy(s_v, s_hbm.at[pl.ds(row0, _SC_G)])
        pltpu.sync_copy(q_v, q_hbm.at[pl.ds(row0, _SC_G)])

    pl.loop(0, n_chunks)(chunk)


def _stage_b(u_flat, idx_flat):
    mesh = plsc.VectorSubcoreMesh(core_axis_name="c", subcore_axis_name="s")
    f = pl.kernel(
        _stage_b_body,
        out_type=[
            jax.ShapeDtypeStruct((B * N, OUT), jnp.float32),
            jax.ShapeDtypeStruct((B * N, OUT), jnp.float32),
            jax.ShapeDtypeStruct((B * N, OUT), jnp.float32),
        ],
        mesh=mesh,
        scratch_types=[
            pltpu.VMEM((_SC_G * K,), jnp.int32),
            pltpu.VMEM((_SC_G * K, OUT), jnp.float32),
            pltpu.VMEM((_SC_G, OUT), jnp.float32),
            pltpu.VMEM((_SC_G, OUT), jnp.float32),
            pltpu.VMEM((_SC_G, OUT), jnp.float32),
            pltpu.SemaphoreType.DMA,
        ],
        compiler_params=pltpu.CompilerParams(use_tc_tiling_on_sc=False),
    )
    return f(u_flat, idx_flat)


_C_ROWS = 2048


def _stage_c1_body(s_ref, q_ref, v_ref, out_ref):
    i = pl.program_id(0)

    @pl.when(i == 0)
    def _init():
        out_ref[...] = jnp.zeros_like(out_ref)

    s = s_ref[...]
    q = q_ref[...]
    v = v_ref[...]
    part = jnp.stack([
        jnp.sum(s, axis=0),
        jnp.sum(q, axis=0),
        jnp.sum(v, axis=0),
        jnp.sum(s * v, axis=0),
        jnp.sum(v * v, axis=0),
        jnp.zeros((OUT,), jnp.float32),
        jnp.zeros((OUT,), jnp.float32),
        jnp.zeros((OUT,), jnp.float32),
    ], axis=0)
    out_ref[...] += part


def _stage_c1(s, q, v):
    grid = ((B * N) // _C_ROWS,)
    return pl.pallas_call(
        _stage_c1_body,
        grid=grid,
        in_specs=[
            pl.BlockSpec((_C_ROWS, OUT), lambda i: (i, 0)),
            pl.BlockSpec((_C_ROWS, OUT), lambda i: (i, 0)),
            pl.BlockSpec((_C_ROWS, OUT), lambda i: (i, 0)),
        ],
        out_specs=pl.BlockSpec((8, OUT), lambda i: (0, 0)),
        out_shape=jax.ShapeDtypeStruct((8, OUT), jnp.float32),
    )(s, q, v)


def _stage_c2_body(m_ref, v_ref, stats_ref, g_ref, b_ref, out_ref):
    cnt = float(B * N * K)
    st = stats_ref[...]
    sum_s, sum_q = st[0:1, :], st[1:2, :]
    sum_v, sum_sv, sum_vv = st[2:3, :], st[3:4, :], st[4:5, :]
    mean = (sum_s + K * sum_v) / cnt
    ey2 = (sum_q + 2.0 * sum_sv + K * sum_vv) / cnt
    var = ey2 - mean * mean
    inv = lax.rsqrt(var + 1e-5)
    ymax = m_ref[...] + v_ref[...]
    t = (ymax - mean) * (inv * g_ref[...]) + b_ref[...]
    out_ref[...] = jnp.where(t >= 0.0, t, 0.2 * t)


def _stage_c2(m, v, stats, gamma, beta):
    grid = ((B * N) // _C_ROWS,)
    return pl.pallas_call(
        _stage_c2_body,
        grid=grid,
        in_specs=[
            pl.BlockSpec((_C_ROWS, OUT), lambda i: (i, 0)),
            pl.BlockSpec((_C_ROWS, OUT), lambda i: (i, 0)),
            pl.BlockSpec((8, OUT), lambda i: (0, 0)),
            pl.BlockSpec((1, OUT), lambda i: (0, 0)),
            pl.BlockSpec((1, OUT), lambda i: (0, 0)),
        ],
        out_specs=pl.BlockSpec((_C_ROWS, OUT), lambda i: (i, 0)),
        out_shape=jax.ShapeDtypeStruct((B * N, OUT), jnp.float32),
    )(m, v, stats, gamma, beta)


def kernel(x, W, gamma, beta):
    w1t = jnp.transpose(W[:, :C])
    wvt = jnp.transpose(W[:, C:] - W[:, :C])
    u, v, idx = _stage_a(x, w1t, wvt)
    u_flat = u.reshape(B * N, OUT)
    idx_flat = idx.reshape(B * N * K)
    m, s, q = _stage_b(u_flat, idx_flat)
    stats = _stage_c1(s, q, v.reshape(B * N, OUT))
    out_t = _stage_c2(m, v.reshape(B * N, OUT), stats,
                      gamma.reshape(1, OUT), beta.reshape(1, OUT))
    return out_t.reshape(B, N, OUT).transpose(0, 2, 1)
